# Initial kernel scaffold; baseline (speedup 1.0000x reference)
import jax
import jax.numpy as jnp
from jax.experimental import pallas as pl


def kernel(q, key_bank, val_bank, W_proj):
    raise NotImplementedError("write your pallas kernel here")


# speedup vs baseline: 1.2553x; 1.2553x over previous
import functools

import jax
import jax.numpy as jnp
from jax import lax
from jax.experimental import pallas as pl
from jax.experimental.pallas import tpu as pltpu
from jax.experimental.pallas import tpu_sc as plsc

NEG_INF = -3.0e38
I32_MAX = 2147483647

_SC_NC = 2
_SC_NS = 16
_SC_NW = _SC_NC * _SC_NS


def _pick_block(m):
    for c in (8000, 4096, 4000, 2048, 2000, 1024, 1000, 512, 256, 128, 64,
              32, 16, 8):
        if m % c == 0:
            return c
    return m


def _topk_merge(vals, ids, k):
    out_v, out_i = [], []
    for _ in range(k):
        m = jnp.max(vals, axis=1, keepdims=True)
        is_max = vals == m
        cid = jnp.min(jnp.where(is_max, ids, I32_MAX), axis=1, keepdims=True)
        out_v.append(m)
        out_i.append(cid)
        vals = jnp.where(is_max & (ids == cid), NEG_INF, vals)
    return jnp.concatenate(out_v, axis=1), jnp.concatenate(out_i, axis=1)


def _scan_body(q_ref, kb_ref, topi_ref, topv_ref, topi_s_ref, *, blk, k):
    i = pl.program_id(0)

    @pl.when(i == 0)
    def _init():
        topv_ref[...] = jnp.full_like(topv_ref, NEG_INF)
        topi_s_ref[...] = jnp.zeros_like(topi_s_ref)

    q = q_ref[...]
    qn = q / (jnp.sqrt(jnp.sum(q * q, axis=1, keepdims=True)) + 1e-6)
    kb = kb_ref[...]
    inv = 1.0 / (jnp.sqrt(jnp.sum(kb * kb, axis=1, keepdims=True)) + 1e-6)
    kn = kb * inv
    s = lax.dot_general(qn.astype(jnp.bfloat16), kn.astype(jnp.bfloat16),
                        (((1,), (1,)), ((), ())),
                        preferred_element_type=jnp.float32)
    ids = i * blk + lax.broadcasted_iota(jnp.int32, s.shape, 1)

    vals = jnp.concatenate([topv_ref[...], s], axis=1)
    allids = jnp.concatenate([topi_s_ref[...], ids], axis=1)
    new_v, new_i = _topk_merge(vals, allids, k)
    topv_ref[...] = new_v
    topi_s_ref[...] = new_i
    topi_ref[...] = new_i


def _topk_scan(q, key_bank, k):
    b, d = q.shape
    m = key_bank.shape[0]
    blk = _pick_block(m)
    grid = m // blk
    return pl.pallas_call(
        functools.partial(_scan_body, blk=blk, k=k),
        grid=(grid,),
        in_specs=[
            pl.BlockSpec((b, d), lambda i: (0, 0)),
            pl.BlockSpec((blk, d), lambda i: (i, 0)),
        ],
        out_specs=pl.BlockSpec((b, k), lambda i: (0, 0)),
        out_shape=jax.ShapeDtypeStruct((b, k), jnp.int32),
        scratch_shapes=[
            pltpu.VMEM((b, k), jnp.float32),
            pltpu.VMEM((b, k), jnp.int32),
        ],
        compiler_params=pltpu.CompilerParams(
            dimension_semantics=("arbitrary",)),
    )(q, key_bank)


def _sc_gather_rows(table, idx):
    n = idx.shape[0]
    d = table.shape[1]
    bpw = n // _SC_NW
    mesh = plsc.VectorSubcoreMesh(core_axis_name="c", subcore_axis_name="s")

    @functools.partial(
        pl.kernel,
        mesh=mesh,
        out_type=jax.ShapeDtypeStruct((n, d), jnp.float32),
        scratch_types=[
            pltpu.VMEM((bpw,), jnp.int32),
            pltpu.VMEM((bpw, d), jnp.float32),
            pltpu.SemaphoreType.DMA,
        ],
    )
    def k(table_hbm, idx_hbm, out_hbm, idx_v, rows_v, sem):
        wid = lax.axis_index("s") * _SC_NC + lax.axis_index("c")
        base = wid * bpw
        pltpu.sync_copy(idx_hbm.at[pl.ds(base, bpw)], idx_v)
        pltpu.async_copy(table_hbm.at[idx_v], rows_v, sem).wait()
        pltpu.sync_copy(rows_v, out_hbm.at[pl.ds(base, bpw)])

    return k(table, idx)


def _proj_body(g0_ref, g1_ref, par_ref, w_ref, o_ref):
    v = jnp.where(par_ref[...] != 0, g1_ref[...], g0_ref[...])
    o_ref[...] = lax.dot_general(
        v, w_ref[...], (((1,), (1,)), ((), ())),
        preferred_element_type=jnp.float32)


def _project(g0, g1, par, w_proj):
    n = g0.shape[0]
    h = w_proj.shape[0]
    return pl.pallas_call(
        _proj_body,
        out_shape=jax.ShapeDtypeStruct((n, h), jnp.float32),
    )(g0, g1, par, w_proj)


def kernel(q, key_bank, val_bank, W_proj):
    b = q.shape[0]
    k = 8
    topi = _topk_scan(q, key_bank, k)
    flat_idx = topi.reshape(b * k)
    vb2 = val_bank.reshape(val_bank.shape[0] // 2, 2 * val_bank.shape[1])
    pairs = _sc_gather_rows(vb2, flat_idx // 2)
    par = (flat_idx % 2).astype(jnp.int32).reshape(b * k, 1)
    g0 = pairs[:, : val_bank.shape[1]]
    g1 = pairs[:, val_bank.shape[1]:]
    dc = _project(g0, g1, par, W_proj)
    return dc.reshape(b, k, W_proj.shape[0])


# speedup vs baseline: 1.7485x; 1.3930x over previous
import functools

import jax
import jax.numpy as jnp
from jax import lax
from jax.experimental import pallas as pl
from jax.experimental.pallas import tpu as pltpu
from jax.experimental.pallas import tpu_sc as plsc

NEG_INF = -3.0e38
I32_MAX = 2147483647
SEG = 32

_SC_NC = 2
_SC_NS = 16
_SC_NW = _SC_NC * _SC_NS


def _pick_block(m):
    for c in (20000, 8000, 4000, 1600, 800, 320, 160, 64, 32):
        if m % c == 0:
            return c
    return m


def _topk_cols(vals, ids, k, axis):
    out_v, out_i = [], []
    for _ in range(k):
        m = jnp.max(vals, axis=axis, keepdims=True)
        is_max = vals == m
        cid = jnp.min(jnp.where(is_max, ids, I32_MAX), axis=axis,
                      keepdims=True)
        out_v.append(m)
        out_i.append(cid)
        vals = jnp.where(is_max & (ids == cid), NEG_INF, vals)
    return (jnp.concatenate(out_v, axis=axis),
            jnp.concatenate(out_i, axis=axis))


def _qn_bf16(q_ref):
    q = q_ref[...]
    qn = q / (jnp.sqrt(jnp.sum(q * q, axis=1, keepdims=True)) + 1e-6)
    return qn.astype(jnp.bfloat16)


def _scan_body(q_ref, kb_ref, topseg_ref, topv_ref, topi_s_ref, *, blk, k):
    i = pl.program_id(0)

    @pl.when(i == 0)
    def _init():
        topv_ref[...] = jnp.full_like(topv_ref, NEG_INF)
        topi_s_ref[...] = jnp.zeros_like(topi_s_ref)

    qn = _qn_bf16(q_ref)
    kb = kb_ref[...]
    inv = 1.0 / (jnp.sqrt(jnp.sum(kb * kb, axis=1, keepdims=True)) + 1e-6)
    kn = kb * inv
    s = lax.dot_general(kn.astype(jnp.bfloat16), qn,
                        (((1,), (1,)), ((), ())),
                        preferred_element_type=jnp.float32)
    nseg = blk // SEG
    segm = jnp.max(s.reshape(nseg, SEG, s.shape[1]), axis=1)
    ids = i * nseg + lax.broadcasted_iota(jnp.int32, segm.shape, 0)

    vals = jnp.concatenate([topv_ref[...], segm], axis=0)
    aids = jnp.concatenate([topi_s_ref[...], ids], axis=0)
    new_v, new_i = _topk_cols(vals, aids, k, axis=0)
    topv_ref[...] = new_v
    topi_s_ref[...] = new_i
    topseg_ref[...] = new_i


def _seg_scan(q, key_bank, k):
    b, d = q.shape
    m = key_bank.shape[0]
    blk = _pick_block(m)
    grid = m // blk
    return pl.pallas_call(
        functools.partial(_scan_body, blk=blk, k=k),
        grid=(grid,),
        in_specs=[
            pl.BlockSpec((b, d), lambda i: (0, 0)),
            pl.BlockSpec((blk, d), lambda i: (i, 0)),
        ],
        out_specs=pl.BlockSpec((k, b), lambda i: (0, 0)),
        out_shape=jax.ShapeDtypeStruct((k, b), jnp.int32),
        scratch_shapes=[
            pltpu.VMEM((k, b), jnp.float32),
            pltpu.VMEM((k, b), jnp.int32),
        ],
        compiler_params=pltpu.CompilerParams(
            dimension_semantics=("arbitrary",)),
    )(q, key_bank)


def _rescore_body(q_ref, cand_ref, cid_ref, topi_ref, *, rows, nc):
    g = pl.program_id(0)
    qn = _qn_bf16(q_ref)
    c = cand_ref[...]
    inv = 1.0 / (jnp.sqrt(jnp.sum(c * c, axis=1, keepdims=True)) + 1e-6)
    cn = c * inv
    s = lax.dot_general(cn.astype(jnp.bfloat16), qn,
                        (((1,), (1,)), ((), ())),
                        preferred_element_type=jnp.float32)
    t = s.reshape(rows, nc, s.shape[1])
    r_loc = lax.broadcasted_iota(jnp.int32, t.shape, 0)
    col = lax.broadcasted_iota(jnp.int32, t.shape, 2)
    sc = jnp.max(jnp.where(col == g * rows + r_loc, t, NEG_INF),
                 axis=2)
    _, top_i = _topk_cols(sc, cid_ref[...], topi_ref.shape[1], axis=1)
    topi_ref[...] = top_i


def _rescore(q, cand, cand_ids, k):
    b, d = q.shape
    nc = cand_ids.shape[1]
    rows = 16
    grid = b // rows
    return pl.pallas_call(
        functools.partial(_rescore_body, rows=rows, nc=nc),
        grid=(grid,),
        in_specs=[
            pl.BlockSpec((b, d), lambda g: (0, 0)),
            pl.BlockSpec((rows * nc, d), lambda g: (g, 0)),
            pl.BlockSpec((rows, nc), lambda g: (g, 0)),
        ],
        out_specs=pl.BlockSpec((rows, k), lambda g: (g, 0)),
        out_shape=jax.ShapeDtypeStruct((b, k), jnp.int32),
        compiler_params=pltpu.CompilerParams(
            dimension_semantics=("arbitrary",)),
    )(q, cand, cand_ids)


def _sc_gather_rows(table, idx):
    n = idx.shape[0]
    d = table.shape[1]
    bpw = n // _SC_NW
    mesh = plsc.VectorSubcoreMesh(core_axis_name="c", subcore_axis_name="s")

    @functools.partial(
        pl.kernel,
        mesh=mesh,
        out_type=jax.ShapeDtypeStruct((n, d), jnp.float32),
        scratch_types=[
            pltpu.VMEM((bpw,), jnp.int32),
            pltpu.VMEM((bpw, d), jnp.float32),
            pltpu.SemaphoreType.DMA,
        ],
    )
    def k(table_hbm, idx_hbm, out_hbm, idx_v, rows_v, sem):
        wid = lax.axis_index("s") * _SC_NC + lax.axis_index("c")
        base = wid * bpw
        pltpu.sync_copy(idx_hbm.at[pl.ds(base, bpw)], idx_v)
        pltpu.async_copy(table_hbm.at[idx_v], rows_v, sem).wait()
        pltpu.sync_copy(rows_v, out_hbm.at[pl.ds(base, bpw)])

    return k(table, idx)


def _proj_body(g0_ref, g1_ref, par_ref, w_ref, o_ref):
    v = jnp.where(par_ref[...] != 0, g1_ref[...], g0_ref[...])
    o_ref[...] = lax.dot_general(
        v, w_ref[...], (((1,), (1,)), ((), ())),
        preferred_element_type=jnp.float32)


def _project(g0, g1, par, w_proj):
    n = g0.shape[0]
    h = w_proj.shape[0]
    return pl.pallas_call(
        _proj_body,
        out_shape=jax.ShapeDtypeStruct((n, h), jnp.float32),
    )(g0, g1, par, w_proj)


def kernel(q, key_bank, val_bank, W_proj):
    b, d = q.shape
    k = 8
    topseg = _seg_scan(q, key_bank, k)
    segs = topseg.T

    ppseg = SEG // 2
    pair_idx = (segs[:, :, None] * ppseg
                + jnp.arange(ppseg, dtype=jnp.int32)[None, None, :]
                ).reshape(-1)
    kb2 = key_bank.reshape(key_bank.shape[0] // 2, 2 * d)
    cand_pairs = _sc_gather_rows(kb2, pair_idx)
    cand = cand_pairs.reshape(b * k * SEG, d)
    cand_ids = (segs[:, :, None] * SEG
                + jnp.arange(SEG, dtype=jnp.int32)[None, None, :]
                ).reshape(b, k * SEG)

    topi = _rescore(q, cand, cand_ids, k)

    flat_idx = topi.reshape(b * k)
    vb2 = val_bank.reshape(val_bank.shape[0] // 2, 2 * d)
    pairs = _sc_gather_rows(vb2, flat_idx // 2)
    par = (flat_idx % 2).astype(jnp.int32).reshape(b * k, 1)
    g0 = pairs[:, :d]
    g1 = pairs[:, d:]
    dc = _project(g0, g1, par, W_proj)
    return dc.reshape(b, k, W_proj.shape[0])
